# baseline (speedup 1.0000x reference)
import functools

import jax
import jax.numpy as jnp
from jax import lax
from jax.experimental import pallas as pl
from jax.experimental.pallas import tpu as pltpu
from jax.experimental.pallas import tpu_sc as plsc

_NC = 2
_NS = 16
_NW = _NC * _NS
_L = 16


def _sc_label_gather(targets_flat, logits_flat, n, c, half):
    nh = n // 2
    words_w = 2 * (nh // _NW)
    mesh = plsc.VectorSubcoreMesh(core_axis_name="c", subcore_axis_name="s",
                                  num_cores=_NC, num_subcores=_NS)

    @functools.partial(
        pl.kernel, mesh=mesh,
        out_type=jax.ShapeDtypeStruct((4 * nh,), jnp.float32),
        scratch_types=[
            pltpu.VMEM((words_w,), jnp.int32),
            pltpu.VMEM((words_w,), jnp.int32),
            pltpu.VMEM((words_w,), jnp.float32),
            pltpu.VMEM((words_w,), jnp.float32),
            pltpu.SemaphoreType.DMA,
        ],
    )
    def k(t_hbm, logits_hbm, out_hbm, t_v, idx_v, g2_v, iou2_v, sem):
        wid = lax.axis_index("s") * _NC + lax.axis_index("c")
        lane = lax.iota(jnp.int32, _L)
        halfk = lane >> 1
        pe = halfk + halfk
        obase = wid * words_w
        wbase = half * n + obase
        rbase = half * (n // 2) + wid * (words_w // 2)
        pltpu.sync_copy(t_hbm.at[pl.ds(wbase, words_w)],
                        t_v.at[pl.ds(0, words_w)])

        def chunk_body(j, carry):
            for b in range(8):
                off = j * 128 + b * _L
                v = t_v[pl.ds(off, _L)]
                clsv = v.at[pe].get(mode="promise_in_bounds")
                iouv = v.at[pe + 1].get(mode="promise_in_bounds")
                row = rbase + (j * 64 + b * 8) + halfk
                safe = jnp.clip(clsv - 1, 0, c - 1)
                idx_v[pl.ds(off, _L)] = row * c + safe
                iou2_v[pl.ds(off, _L)] = jnp.where(
                    clsv >= 1, iouv.astype(jnp.float32), -1.0)
            pltpu.async_copy(
                logits_hbm.at[idx_v.at[pl.ds(j * 128, 128)]],
                g2_v.at[pl.ds(j * 128, 128)], sem)
            return carry

        lax.fori_loop(0, words_w // 128, chunk_body, 0)
        pltpu.make_async_copy(logits_hbm.at[pl.ds(0, words_w)], g2_v,
                              sem).wait()
        pltpu.sync_copy(g2_v, out_hbm.at[pl.ds(obase, words_w)])
        pltpu.sync_copy(iou2_v, out_hbm.at[pl.ds(2 * nh + obase, words_w)])

    return k(targets_flat, logits_flat)


def _dense_body(x_ref, out_ref):
    i = pl.program_id(0)
    l = x_ref[:]
    th = jnp.tanh(0.5 * l)
    q = jnp.maximum(0.5 - 0.5 * th, 1e-37)
    s = jnp.sum(jnp.log(q) - q)

    @pl.when(i == 0)
    def _():
        out_ref[...] = jnp.zeros_like(out_ref)

    out_ref[...] += s.reshape(1, 1)


def _corr_half(l, ie, invm):
    lab = jnp.maximum(ie, 0.0) * invm
    th = jnp.tanh(0.5 * l)
    q = jnp.maximum(0.5 - 0.5 * th, 1e-37)
    p = 1.0 - q
    sp = -jnp.log(q)
    c_neg = 0.75 * lab * (1.0 - sp)
    c_pos = 0.25 * lab * (sp - l - 1.0) + p - 0.75 * sp
    corr = jnp.where(p <= lab, c_pos, c_neg)
    return jnp.sum(corr)


def _corr_body(s0_ref, ga_ref, ia_ref, gb_ref, ib_ref, out_ref, *, count):
    m = jnp.maximum(jnp.max(ia_ref[:]), jnp.max(ib_ref[:]))
    invm = 1.0 / m
    cs = _corr_half(ga_ref[:], ia_ref[:], invm) + \
        _corr_half(gb_ref[:], ib_ref[:], invm)
    out_ref[...] = (-0.75) * (s0_ref[...] + count) + (0.5 * cs).reshape(1, 1)


def kernel(logits, targets):
    n, c = logits.shape

    tf, lf = targets.reshape(-1), logits.reshape(-1)
    oa = _sc_label_gather(tf, lf, n, c, 0)
    ob = _sc_label_gather(tf, lf, n, c, 1)
    g2a, iou2a = oa[:n], oa[n:]
    g2b, iou2b = ob[:n], ob[n:]

    bb = 4096
    s0 = pl.pallas_call(
        _dense_body,
        grid=(n // bb,),
        in_specs=[pl.BlockSpec((bb, c), lambda i: (i, 0))],
        out_specs=pl.BlockSpec((1, 1), lambda i: (0, 0)),
        out_shape=jax.ShapeDtypeStruct((1, 1), jnp.float32),
    )(logits)

    nr = n // 128
    out = pl.pallas_call(
        functools.partial(_corr_body, count=float(n * c)),
        in_specs=[
            pl.BlockSpec((1, 1), lambda: (0, 0)),
            pl.BlockSpec((nr, 128), lambda: (0, 0)),
            pl.BlockSpec((nr, 128), lambda: (0, 0)),
            pl.BlockSpec((nr, 128), lambda: (0, 0)),
            pl.BlockSpec((nr, 128), lambda: (0, 0)),
        ],
        out_specs=pl.BlockSpec((1, 1), lambda: (0, 0)),
        out_shape=jax.ShapeDtypeStruct((1, 1), jnp.float32),
    )(s0, g2a.reshape(nr, 128), iou2a.reshape(nr, 128),
      g2b.reshape(nr, 128), iou2b.reshape(nr, 128))
    return out[0, 0]


# speedup vs baseline: 1.0080x; 1.0080x over previous
import functools

import jax
import jax.numpy as jnp
from jax import lax
from jax.experimental import pallas as pl
from jax.experimental.pallas import tpu as pltpu
from jax.experimental.pallas import tpu_sc as plsc

_NC = 2
_NS = 16
_NW = _NC * _NS
_L = 16


def _sc_label_gather(targets_flat, logits_flat, n, c, half):
    nh = n // 2
    words_w = 2 * (nh // _NW)
    mesh = plsc.VectorSubcoreMesh(core_axis_name="c", subcore_axis_name="s",
                                  num_cores=_NC, num_subcores=_NS)

    @functools.partial(
        pl.kernel, mesh=mesh,
        out_type=jax.ShapeDtypeStruct((4 * nh,), jnp.float32),
        scratch_types=[
            pltpu.VMEM((words_w,), jnp.int32),
            pltpu.VMEM((words_w,), jnp.int32),
            pltpu.VMEM((words_w,), jnp.float32),
            pltpu.VMEM((words_w,), jnp.float32),
            pltpu.SemaphoreType.DMA,
        ],
    )
    def k(t_hbm, logits_hbm, out_hbm, t_v, idx_v, g2_v, iou2_v, sem):
        wid = lax.axis_index("s") * _NC + lax.axis_index("c")
        lane = lax.iota(jnp.int32, _L)
        halfk = lane >> 1
        pe = halfk + halfk
        obase = wid * words_w
        wbase = half * n + obase
        rbase = half * (n // 2) + wid * (words_w // 2)
        pltpu.sync_copy(t_hbm.at[pl.ds(wbase, words_w)],
                        t_v.at[pl.ds(0, words_w)])

        def chunk_body(j, carry):
            for b in range(8):
                off = j * 128 + b * _L
                v = t_v[pl.ds(off, _L)]
                clsv = v.at[pe].get(mode="promise_in_bounds")
                iouv = v.at[pe + 1].get(mode="promise_in_bounds")
                row = rbase + (j * 64 + b * 8) + halfk
                safe = jnp.clip(clsv - 1, 0, c - 1)
                idx_v[pl.ds(off, _L)] = row * c + safe
                iou2_v[pl.ds(off, _L)] = jnp.where(
                    clsv >= 1, iouv.astype(jnp.float32), -1.0)
            pltpu.async_copy(
                logits_hbm.at[idx_v.at[pl.ds(j * 128, 128)]],
                g2_v.at[pl.ds(j * 128, 128)], sem)
            return carry

        lax.fori_loop(0, words_w // 128, chunk_body, 0)
        pltpu.make_async_copy(logits_hbm.at[pl.ds(0, words_w)], g2_v,
                              sem).wait()
        pltpu.sync_copy(g2_v, out_hbm.at[pl.ds(obase, words_w)])
        pltpu.sync_copy(iou2_v, out_hbm.at[pl.ds(2 * nh + obase, words_w)])

    return k(targets_flat, logits_flat)


def _dense_body(x_ref, out_ref):
    i = pl.program_id(0)
    l = x_ref[:]
    th = jnp.tanh(0.5 * l)
    q = jnp.maximum(0.5 - 0.5 * th, 1e-37)
    s = jnp.sum(jnp.log(q) - q)

    @pl.when(i == 0)
    def _():
        out_ref[...] = jnp.zeros_like(out_ref)

    out_ref[...] += s.reshape(1, 1)


def _corr_half(l, ie, invm):
    lab = jnp.maximum(ie, 0.0) * invm
    th = jnp.tanh(0.5 * l)
    q = jnp.maximum(0.5 - 0.5 * th, 1e-37)
    p = 1.0 - q
    sp = -jnp.log(q)
    c_neg = 0.75 * lab * (1.0 - sp)
    c_pos = 0.25 * lab * (sp - l - 1.0) + p - 0.75 * sp
    corr = jnp.where(p <= lab, c_pos, c_neg)
    return jnp.sum(corr)


def _corr_body(s0_ref, pa_ref, pb_ref, out_ref, *, count, nr):
    ga, ia = pa_ref[:nr, :], pa_ref[nr:, :]
    gb, ib = pb_ref[:nr, :], pb_ref[nr:, :]
    m = jnp.maximum(jnp.max(ia), jnp.max(ib))
    invm = 1.0 / m
    cs = _corr_half(ga, ia, invm) + _corr_half(gb, ib, invm)
    out_ref[...] = (-0.75) * (s0_ref[...] + count) + (0.5 * cs).reshape(1, 1)


def kernel(logits, targets):
    n, c = logits.shape

    tf, lf = targets.reshape(-1), logits.reshape(-1)
    oa = _sc_label_gather(tf, lf, n, c, 0)
    ob = _sc_label_gather(tf, lf, n, c, 1)

    bb = 4096
    s0 = pl.pallas_call(
        _dense_body,
        grid=(n // bb,),
        in_specs=[pl.BlockSpec((bb, c), lambda i: (i, 0))],
        out_specs=pl.BlockSpec((1, 1), lambda i: (0, 0)),
        out_shape=jax.ShapeDtypeStruct((1, 1), jnp.float32),
    )(logits)

    nr = n // 128
    out = pl.pallas_call(
        functools.partial(_corr_body, count=float(n * c), nr=nr),
        in_specs=[
            pl.BlockSpec((1, 1), lambda: (0, 0)),
            pl.BlockSpec((2 * nr, 128), lambda: (0, 0)),
            pl.BlockSpec((2 * nr, 128), lambda: (0, 0)),
        ],
        out_specs=pl.BlockSpec((1, 1), lambda: (0, 0)),
        out_shape=jax.ShapeDtypeStruct((1, 1), jnp.float32),
    )(s0, oa.reshape(2 * nr, 128), ob.reshape(2 * nr, 128))
    return out[0, 0]


# speedup vs baseline: 1.0097x; 1.0018x over previous
import functools

import jax
import jax.numpy as jnp
from jax import lax
from jax.experimental import pallas as pl
from jax.experimental.pallas import tpu as pltpu
from jax.experimental.pallas import tpu_sc as plsc

_NC = 2
_NS = 16
_NW = _NC * _NS
_L = 16


def _sc_label_gather(targets_flat, logits_flat, n, c):
    nh = n // 2
    words_w = 2 * (nh // _NW)
    mesh = plsc.VectorSubcoreMesh(core_axis_name="c", subcore_axis_name="s",
                                  num_cores=_NC, num_subcores=_NS)

    @functools.partial(
        pl.kernel, mesh=mesh,
        out_type=jax.ShapeDtypeStruct((4 * n,), jnp.float32),
        scratch_types=[
            pltpu.VMEM((words_w,), jnp.int32),
            pltpu.VMEM((words_w,), jnp.int32),
            pltpu.VMEM((words_w,), jnp.float32),
            pltpu.VMEM((words_w,), jnp.float32),
            pltpu.SemaphoreType.DMA,
        ],
    )
    def k(t_hbm, logits_hbm, out_hbm, t_v, idx_v, g2_v, iou2_v, sem):
        wid = lax.axis_index("s") * _NC + lax.axis_index("c")
        lane = lax.iota(jnp.int32, _L)
        halfk = lane >> 1
        pe = halfk + halfk

        for half in range(2):
            obase = half * n + wid * words_w
            rbase = half * nh + wid * (words_w // 2)
            pltpu.sync_copy(t_hbm.at[pl.ds(obase, words_w)],
                            t_v.at[pl.ds(0, words_w)])

            def chunk_body(j, carry):
                for b in range(8):
                    off = j * 128 + b * _L
                    v = t_v[pl.ds(off, _L)]
                    clsv = v.at[pe].get(mode="promise_in_bounds")
                    iouv = v.at[pe + 1].get(mode="promise_in_bounds")
                    row = rbase + (j * 64 + b * 8) + halfk
                    safe = jnp.clip(clsv - 1, 0, c - 1)
                    idx_v[pl.ds(off, _L)] = row * c + safe
                    iou2_v[pl.ds(off, _L)] = jnp.where(
                        clsv >= 1, iouv.astype(jnp.float32), -1.0)
                pltpu.async_copy(
                    logits_hbm.at[idx_v.at[pl.ds(j * 128, 128)]],
                    g2_v.at[pl.ds(j * 128, 128)], sem)
                return carry

            lax.fori_loop(0, words_w // 128, chunk_body, 0)
            pltpu.make_async_copy(logits_hbm.at[pl.ds(0, words_w)], g2_v,
                                  sem).wait()
            pltpu.sync_copy(g2_v, out_hbm.at[pl.ds(obase, words_w)])
            pltpu.sync_copy(iou2_v, out_hbm.at[pl.ds(2 * n + obase, words_w)])

    return k(targets_flat, logits_flat)


def _dense_body(x_ref, out_ref):
    i = pl.program_id(0)
    l = x_ref[:]
    th = jnp.tanh(0.5 * l)
    q = jnp.maximum(0.5 - 0.5 * th, 1e-37)
    s = jnp.sum(jnp.log(q) - q)

    @pl.when(i == 0)
    def _():
        out_ref[...] = jnp.zeros_like(out_ref)

    out_ref[...] += s.reshape(1, 1)


def _corr_half(l, ie, invm):
    lab = jnp.maximum(ie, 0.0) * invm
    th = jnp.tanh(0.5 * l)
    q = jnp.maximum(0.5 - 0.5 * th, 1e-37)
    p = 1.0 - q
    sp = -jnp.log(q)
    c_neg = 0.75 * lab * (1.0 - sp)
    c_pos = 0.25 * lab * (sp - l - 1.0) + p - 0.75 * sp
    corr = jnp.where(p <= lab, c_pos, c_neg)
    return jnp.sum(corr)


def _corr_body(s0_ref, pk_ref, out_ref, *, count, nr):
    g2, i2 = pk_ref[:2 * nr, :], pk_ref[2 * nr:, :]
    m = jnp.max(i2)
    invm = 1.0 / m
    cs = _corr_half(g2, i2, invm)
    out_ref[...] = (-0.75) * (s0_ref[...] + count) + (0.5 * cs).reshape(1, 1)


def kernel(logits, targets):
    n, c = logits.shape

    tf, lf = targets.reshape(-1), logits.reshape(-1)
    pk = _sc_label_gather(tf, lf, n, c)

    bb = 4096
    s0 = pl.pallas_call(
        _dense_body,
        grid=(n // bb,),
        in_specs=[pl.BlockSpec((bb, c), lambda i: (i, 0))],
        out_specs=pl.BlockSpec((1, 1), lambda i: (0, 0)),
        out_shape=jax.ShapeDtypeStruct((1, 1), jnp.float32),
    )(logits)

    nr = n // 128
    out = pl.pallas_call(
        functools.partial(_corr_body, count=float(n * c), nr=nr),
        in_specs=[
            pl.BlockSpec((1, 1), lambda: (0, 0)),
            pl.BlockSpec((4 * nr, 128), lambda: (0, 0)),
        ],
        out_specs=pl.BlockSpec((1, 1), lambda: (0, 0)),
        out_shape=jax.ShapeDtypeStruct((1, 1), jnp.float32),
    )(s0, pk.reshape(4 * nr, 128))
    return out[0, 0]


# speedup vs baseline: 1.4418x; 1.4279x over previous
import functools

import jax
import jax.numpy as jnp
from jax import lax
from jax.experimental import pallas as pl
from jax.experimental.pallas import tpu as pltpu
from jax.experimental.pallas import tpu_sc as plsc

_NC = 2
_NS = 16
_NW = _NC * _NS
_L = 16


def _sc_gather(cls_flat, logits_flat, n, c):
    rows_w = n // _NW
    chunks = rows_w // 128
    mesh = plsc.VectorSubcoreMesh(core_axis_name="c", subcore_axis_name="s",
                                  num_cores=_NC, num_subcores=_NS)

    @functools.partial(
        pl.kernel, mesh=mesh,
        out_type=jax.ShapeDtypeStruct((n,), jnp.float32),
        scratch_types=[
            pltpu.VMEM((rows_w,), jnp.int32),
            pltpu.VMEM((rows_w,), jnp.int32),
            pltpu.VMEM((rows_w,), jnp.float32),
            pltpu.SemaphoreType.DMA,
        ],
    )
    def k(cls_hbm, logits_hbm, g_hbm, cls_v, idx_v, g_v, sem):
        wid = lax.axis_index("s") * _NC + lax.axis_index("c")
        base = wid * rows_w
        pltpu.sync_copy(cls_hbm.at[pl.ds(base, rows_w)], cls_v)

        def chunk_body(j, carry):
            for b in range(8):
                off = j * 128 + b * _L
                lane = lax.iota(jnp.int32, _L)
                v = cls_v[pl.ds(off, _L)]
                safe = jnp.clip(v - 1, 0, c - 1)
                idx_v[pl.ds(off, _L)] = (base + off + lane) * c + safe
            pltpu.async_copy(
                logits_hbm.at[idx_v.at[pl.ds(j * 128, 128)]],
                g_v.at[pl.ds(j * 128, 128)], sem)
            return carry

        lax.fori_loop(0, chunks, chunk_body, 0)
        pltpu.make_async_copy(logits_hbm.at[pl.ds(0, rows_w)], g_v, sem).wait()
        pltpu.sync_copy(g_v, g_hbm.at[pl.ds(base, rows_w)])

    return k(cls_flat, logits_flat)


def _dense_body(x_ref, out_ref):
    i = pl.program_id(0)
    l = x_ref[:]
    th = jnp.tanh(0.5 * l)
    q = jnp.maximum(0.5 - 0.5 * th, 1e-37)
    s = jnp.sum(jnp.log(q) - q)

    @pl.when(i == 0)
    def _():
        out_ref[...] = jnp.zeros_like(out_ref)

    out_ref[...] += s.reshape(1, 1)


def _corr_body(s0_ref, g_ref, clsr_ref, iour_ref, out_ref, *, count):
    valid = clsr_ref[:] >= 1
    m = jnp.max(jnp.where(valid, iour_ref[:], 0)).astype(jnp.float32)
    lab = jnp.where(valid, iour_ref[:].astype(jnp.float32) * (1.0 / m), 0.0)
    l = g_ref[:]
    th = jnp.tanh(0.5 * l)
    q = jnp.maximum(0.5 - 0.5 * th, 1e-37)
    p = 1.0 - q
    sp = -jnp.log(q)
    c_neg = 0.75 * lab * (1.0 - sp)
    c_pos = 0.25 * lab * (sp - l - 1.0) + p - 0.75 * sp
    corr = jnp.where(p <= lab, c_pos, c_neg)
    out_ref[...] = (-0.75) * (s0_ref[...] + count) + jnp.sum(corr).reshape(1, 1)


def kernel(logits, targets):
    n, c = logits.shape
    cls_flat = targets[:, 0]
    clsr = cls_flat.reshape(n // 128, 128)
    iour = targets[:, 1].reshape(n // 128, 128)

    g = _sc_gather(cls_flat, logits.reshape(-1), n, c)

    bb = 4096
    s0 = pl.pallas_call(
        _dense_body,
        grid=(n // bb,),
        in_specs=[pl.BlockSpec((bb, c), lambda i: (i, 0))],
        out_specs=pl.BlockSpec((1, 1), lambda i: (0, 0)),
        out_shape=jax.ShapeDtypeStruct((1, 1), jnp.float32),
    )(logits)

    nr = n // 128
    out = pl.pallas_call(
        functools.partial(_corr_body, count=float(n * c)),
        in_specs=[
            pl.BlockSpec((1, 1), lambda: (0, 0)),
            pl.BlockSpec((nr, 128), lambda: (0, 0)),
            pl.BlockSpec((nr, 128), lambda: (0, 0)),
            pl.BlockSpec((nr, 128), lambda: (0, 0)),
        ],
        out_specs=pl.BlockSpec((1, 1), lambda: (0, 0)),
        out_shape=jax.ShapeDtypeStruct((1, 1), jnp.float32),
    )(s0, g.reshape(nr, 128), clsr, iour)
    return out[0, 0]
